# baseline (speedup 1.0000x reference)
import functools

import jax
import jax.numpy as jnp
from jax import lax
from jax.experimental import pallas as pl
from jax.experimental.pallas import tpu as pltpu
from jax.experimental.pallas import tpu_sc as plsc

B = 16384
D = 64
H = 2 * D
V = 100000
NC = 2
NS = 16
NW = NC * NS
BPW = B // NW
CH = BPW // 2
L = 16
TB = 4096
NB = B // TB
AVG_R = 3.5
LAM = 0.001

WU = 256
NWIN = V // WU
NW_LO = NWIN // NW
NW_EXTRA = NWIN - NW_LO * NW
TAILU = V - NWIN * WU
TAIL_WID = NW_EXTRA


def _sc_transpose(euT, eiT, eu_tail, ei_tail):
    mesh = plsc.VectorSubcoreMesh(core_axis_name="c", subcore_axis_name="s")

    @functools.partial(
        pl.kernel,
        mesh=mesh,
        compiler_params=pltpu.CompilerParams(needs_layout_passes=False),
        out_type=(
            jax.ShapeDtypeStruct((V // 2, 128), jnp.float32),
            jax.ShapeDtypeStruct((V // 2, 128), jnp.float32),
        ),
        scratch_types=(
            pltpu.VMEM((D, WU), jnp.float32),
            pltpu.VMEM((D, WU), jnp.float32),
            pltpu.VMEM((WU // 2, 128), jnp.float32),
            pltpu.VMEM((WU // 2, 128), jnp.float32),
            pltpu.SemaphoreType.DMA,
            pltpu.SemaphoreType.DMA,
        ),
    )
    def tkernel(euT_h, eiT_h, eut_h, eit_h, eu2_h, ei2_h,
                win_u, win_i, out_u, out_i, sem_u, sem_i):
        wid = lax.axis_index("s") * NC + lax.axis_index("c")
        iota = lax.iota(jnp.int32, L)
        ridx = [lax.shift_right_logical(iota + L * ub, 1)
                for ub in range(WU // L)]
        cbase = [((iota + L * ub) & 1) * D for ub in range(WU // L)]
        uvecs = [iota + L * ub for ub in range(WU // L)]

        def transpose_win(win, out):
            def feat(d, carry):
                dvec, cvecs = carry
                for ub in range(WU // L):
                    v = plsc.load_gather(win, [dvec, uvecs[ub]])
                    plsc.store_scatter(out, [ridx[ub], cvecs[ub]], v)
                return (dvec + 1, [cv + 1 for cv in cvecs])

            lax.fori_loop(0, D, feat, (iota * 0, list(cbase)), unroll=2)

        def window(k, carry):
            w = wid + k * NW

            @pl.when(w < NWIN)
            def _do():
                col = pl.multiple_of(w * WU, WU)
                prow = pl.multiple_of(w * (WU // 2), WU // 2)
                cu = pltpu.async_copy(euT_h.at[:, pl.ds(col, WU)],
                                      win_u, sem_u)
                ci = pltpu.async_copy(eiT_h.at[:, pl.ds(col, WU)],
                                      win_i, sem_i)
                cu.wait()
                transpose_win(win_u, out_u)
                ci.wait()
                transpose_win(win_i, out_i)
                pltpu.sync_copy(out_u, eu2_h.at[pl.ds(prow, WU // 2)])
                pltpu.sync_copy(out_i, ei2_h.at[pl.ds(prow, WU // 2)])

            return carry

        lax.fori_loop(0, NW_LO + 1, window, 0)

        @pl.when(wid == TAIL_WID)
        def _tail():
            prow = NWIN * (WU // 2)
            pltpu.sync_copy(eut_h, eu2_h.at[pl.ds(prow, TAILU // 2)])
            pltpu.sync_copy(eit_h, ei2_h.at[pl.ds(prow, TAILU // 2)])

    return tkernel(euT, eiT, eu_tail, ei_tail)


def _sc_gather(user0, item_i0, ratings, eu2, ei2, user_bias, item_bias):
    mesh = plsc.VectorSubcoreMesh(core_axis_name="c", subcore_axis_name="s")

    @functools.partial(
        pl.kernel,
        mesh=mesh,
        out_type=(
            jax.ShapeDtypeStruct((2 * B, 128), jnp.float32),
            jax.ShapeDtypeStruct((B,), jnp.float32),
        ),
        scratch_types=(
            pltpu.VMEM((BPW,), jnp.int32),
            pltpu.VMEM((BPW,), jnp.int32),
            pltpu.VMEM((BPW,), jnp.int32),
            pltpu.VMEM((BPW,), jnp.int32),
            pltpu.VMEM((CH, 128), jnp.float32),
            pltpu.VMEM((CH, 128), jnp.float32),
            pltpu.VMEM((BPW,), jnp.float32),
            pltpu.VMEM((BPW,), jnp.float32),
            pltpu.VMEM((BPW,), jnp.float32),
            pltpu.VMEM((BPW,), jnp.float32),
            pltpu.SemaphoreType.DMA,
            pltpu.SemaphoreType.DMA,
            pltpu.SemaphoreType.DMA,
            pltpu.SemaphoreType.DMA,
            pltpu.SemaphoreType.DMA,
        ),
    )
    def gather_kernel(u0_hbm, i0_hbm, rat_hbm, eu_hbm, ei_hbm, ubt_hbm,
                      ibt_hbm,
                      rows_out, c_out,
                      uidx_v, iidx_v, upair_v, ipair_v,
                      ubuf, ibuf, ubv, ibv, ratv, cv,
                      sem_u, sem_i, sem_ub, sem_ib, sem_r):
        wid = lax.axis_index("s") * NC + lax.axis_index("c")
        base = pl.multiple_of(wid * BPW, BPW)
        pltpu.sync_copy(u0_hbm.at[pl.ds(base, BPW)], uidx_v)
        pltpu.sync_copy(i0_hbm.at[pl.ds(base, BPW)], iidx_v)
        cub = pltpu.async_copy(ubt_hbm.at[uidx_v], ubv, sem_ub)
        cib = pltpu.async_copy(ibt_hbm.at[iidx_v], ibv, sem_ib)
        crat = pltpu.async_copy(rat_hbm.at[pl.ds(base, BPW)], ratv, sem_r)

        def pair_chunk(k, carry):
            s = pl.multiple_of(k * L, L)
            upair_v[pl.ds(s, L)] = lax.shift_right_logical(
                uidx_v[pl.ds(s, L)], 1)
            ipair_v[pl.ds(s, L)] = lax.shift_right_logical(
                iidx_v[pl.ds(s, L)], 1)
            return carry

        lax.fori_loop(0, BPW // L, pair_chunk, 0, unroll=4)

        cu0 = pltpu.async_copy(eu_hbm.at[upair_v.at[pl.ds(0, CH)]],
                               ubuf, sem_u)
        ci0 = pltpu.async_copy(ei_hbm.at[ipair_v.at[pl.ds(0, CH)]],
                               ibuf, sem_i)
        cu0.wait()
        pltpu.sync_copy(ubuf, rows_out.at[pl.ds(base, CH)])
        cu1 = pltpu.async_copy(eu_hbm.at[upair_v.at[pl.ds(CH, CH)]],
                               ubuf, sem_u)
        ci0.wait()
        pltpu.sync_copy(ibuf, rows_out.at[pl.ds(B + base, CH)])
        ci1 = pltpu.async_copy(ei_hbm.at[ipair_v.at[pl.ds(CH, CH)]],
                               ibuf, sem_i)
        cub.wait()
        cib.wait()
        crat.wait()

        def c_chunk(k, carry):
            s = pl.multiple_of(k * L, L)
            cv[pl.ds(s, L)] = (ubv[pl.ds(s, L)] + ibv[pl.ds(s, L)]
                               + AVG_R - ratv[pl.ds(s, L)])
            return carry

        lax.fori_loop(0, BPW // L, c_chunk, 0, unroll=4)
        pltpu.sync_copy(cv, c_out.at[pl.ds(base, BPW)])

        cu1.wait()
        pltpu.sync_copy(ubuf, rows_out.at[pl.ds(base + CH, CH)])
        ci1.wait()
        pltpu.sync_copy(ibuf, rows_out.at[pl.ds(B + base + CH, CH)])

    return gather_kernel(user0, item_i0, ratings, eu2, ei2,
                         user_bias, item_bias)


def _tc_body(u_ref, it_ref, u0_ref, i0_ref, c_ref,
             W1_ref, b1_ref, W2_ref, b2_ref, part_ref):
    W1 = W1_ref[...]
    b1 = b1_ref[...]
    W2 = W2_ref[...]
    b2 = b2_ref[...]

    def mlp(x):
        h = jnp.dot(x, W1, preferred_element_type=jnp.float32) + b1
        h = jnp.where(h >= 0, h, 0.1 * h)
        return jnp.dot(h, W2, preferred_element_type=jnp.float32) + b2

    xu2 = u_ref[...]
    xi2 = it_ref[...]
    pu = (u0_ref[...] & 1) == 0
    pi = (i0_ref[...] & 1) == 0
    fu_l = mlp(xu2[:, :D])
    fu_r = mlp(xu2[:, D:])
    fi_l = mlp(xi2[:, :D])
    fi_r = mlp(xi2[:, D:])
    dll = jnp.sum(fu_l * fi_l, axis=1)
    dlr = jnp.sum(fu_l * fi_r, axis=1)
    drl = jnp.sum(fu_r * fi_l, axis=1)
    drr = jnp.sum(fu_r * fi_r, axis=1)
    dots = jnp.where(pu, jnp.where(pi, dll, dlr), jnp.where(pi, drl, drr))
    err = dots + c_ref[...]
    squ = jnp.where(pu, jnp.sum(fu_l * fu_l, axis=1),
                    jnp.sum(fu_r * fu_r, axis=1))
    sqi = jnp.where(pi, jnp.sum(fi_l * fi_l, axis=1),
                    jnp.sum(fi_r * fi_r, axis=1))
    i = pl.program_id(0)
    part_ref[i, 0] = jnp.sum(err * err)
    part_ref[i, 1] = jnp.sum(squ)
    part_ref[i, 2] = jnp.sum(sqi)


def _tc_loss(rows2, u0, i0, c, W1, b1, W2, b2):
    return pl.pallas_call(
        _tc_body,
        grid=(NB,),
        in_specs=[
            pl.BlockSpec((TB, 128), lambda i: (i, 0)),
            pl.BlockSpec((TB, 128), lambda i: (NB + i, 0)),
            pl.BlockSpec((TB,), lambda i: (i,)),
            pl.BlockSpec((TB,), lambda i: (i,)),
            pl.BlockSpec((TB,), lambda i: (i,)),
            pl.BlockSpec((D, H), lambda i: (0, 0)),
            pl.BlockSpec((1, H), lambda i: (0, 0)),
            pl.BlockSpec((H, D), lambda i: (0, 0)),
            pl.BlockSpec((1, D), lambda i: (0, 0)),
        ],
        out_specs=pl.BlockSpec(memory_space=pltpu.SMEM),
        out_shape=jax.ShapeDtypeStruct((NB, 3), jnp.float32),
    )(rows2, rows2, u0, i0, c, W1, b1, W2, b2)


def kernel(user0, item_i0, ratings, embed_user, embed_item,
           W1, b1, W2, b2, user_bias, item_bias):
    u0 = user0.astype(jnp.int32)
    i0 = item_i0.astype(jnp.int32)
    eu_tail = embed_user[NWIN * WU:].reshape(TAILU // 2, 128)
    ei_tail = embed_item[NWIN * WU:].reshape(TAILU // 2, 128)
    eu2, ei2 = _sc_transpose(embed_user.T, embed_item.T, eu_tail, ei_tail)
    rows2, c = _sc_gather(u0, i0, ratings.astype(jnp.float32), eu2, ei2,
                          user_bias[:, 0], item_bias[:, 0])
    parts = _tc_loss(rows2, u0, i0, c,
                     W1, b1.reshape(1, H), W2, b2.reshape(1, D))
    sums = jnp.sum(parts, axis=0)
    loss2 = sums[0] / B
    l2 = LAM * (sums[1] / (B * D)) + LAM * (sums[2] / (B * D))
    loss = loss2 + l2
    z = jnp.float32(0.0)
    return (loss, loss2, z, z, z, z)


# speedup vs baseline: 3.4001x; 3.4001x over previous
import functools

import jax
import jax.numpy as jnp
from jax import lax
from jax.experimental import pallas as pl
from jax.experimental.pallas import tpu as pltpu
from jax.experimental.pallas import tpu_sc as plsc
from jax.experimental.layout import Format, Layout, with_layout_constraint

B = 16384
D = 64
H = 2 * D
V = 100000
NC = 2
NS = 16
NW = NC * NS
BPW = B // NW
L = 16
TB = 4096
NB = B // TB
AVG_R = 3.5
LAM = 0.001


def _sc_gather(user0, item_i0, ratings, eu8, ei8, user_bias, item_bias):
    mesh = plsc.VectorSubcoreMesh(core_axis_name="c", subcore_axis_name="s")

    @functools.partial(
        pl.kernel,
        mesh=mesh,
        compiler_params=pltpu.CompilerParams(use_tc_tiling_on_sc=False),
        out_type=(
            jax.ShapeDtypeStruct((B, 128), jnp.float32),
            jax.ShapeDtypeStruct((B,), jnp.float32),
        ),
        scratch_types=(
            pltpu.VMEM((BPW,), jnp.int32),
            pltpu.VMEM((BPW,), jnp.int32),
            pltpu.VMEM((BPW, D), jnp.float32),
            pltpu.VMEM((BPW, D), jnp.float32),
            pltpu.VMEM((BPW,), jnp.float32),
            pltpu.VMEM((BPW,), jnp.float32),
            pltpu.VMEM((BPW,), jnp.float32),
            pltpu.VMEM((BPW,), jnp.float32),
            pltpu.SemaphoreType.DMA,
            pltpu.SemaphoreType.DMA,
            pltpu.SemaphoreType.DMA,
            pltpu.SemaphoreType.DMA,
            pltpu.SemaphoreType.DMA,
        ),
    )
    def gather_kernel(u0_hbm, i0_hbm, rat_hbm, eu_hbm, ei_hbm, ubt_hbm,
                      ibt_hbm,
                      rows_out, c_out,
                      uidx_v, iidx_v, urows_v, irows_v, ubv, ibv, ratv, cv,
                      sem_u, sem_i, sem_ub, sem_ib, sem_r):
        wid = lax.axis_index("s") * NC + lax.axis_index("c")
        base = pl.multiple_of(wid * BPW, BPW)
        pltpu.sync_copy(u0_hbm.at[pl.ds(base, BPW)], uidx_v)
        pltpu.sync_copy(i0_hbm.at[pl.ds(base, BPW)], iidx_v)
        cu = pltpu.async_copy(eu_hbm.at[uidx_v], urows_v, sem_u)
        ci = pltpu.async_copy(ei_hbm.at[iidx_v], irows_v, sem_i)
        cub = pltpu.async_copy(ubt_hbm.at[uidx_v], ubv, sem_ub)
        cib = pltpu.async_copy(ibt_hbm.at[iidx_v], ibv, sem_ib)
        crat = pltpu.async_copy(rat_hbm.at[pl.ds(base, BPW)], ratv, sem_r)
        cub.wait()
        cib.wait()
        crat.wait()

        def c_chunk(k, carry):
            s = pl.multiple_of(k * L, L)
            cv[pl.ds(s, L)] = (ubv[pl.ds(s, L)] + ibv[pl.ds(s, L)]
                               + AVG_R - ratv[pl.ds(s, L)])
            return carry

        lax.fori_loop(0, BPW // L, c_chunk, 0, unroll=4)
        pltpu.sync_copy(cv, c_out.at[pl.ds(base, BPW)])
        cu.wait()
        pltpu.sync_copy(urows_v,
                        rows_out.at[pl.ds(base, BPW), pl.ds(0, D)])
        ci.wait()
        pltpu.sync_copy(irows_v,
                        rows_out.at[pl.ds(base, BPW), pl.ds(D, D)])

    return gather_kernel(user0, item_i0, ratings, eu8, ei8,
                         user_bias, item_bias)


def _tc_body(x_ref, c_ref, W1_ref, b1_ref, W2_ref, b2_ref, part_ref):
    W1 = W1_ref[...]
    b1 = b1_ref[...]
    W2 = W2_ref[...]
    b2 = b2_ref[...]

    def mlp(x):
        h = jnp.dot(x, W1, preferred_element_type=jnp.float32) + b1
        h = jnp.where(h >= 0, h, 0.1 * h)
        return jnp.dot(h, W2, preferred_element_type=jnp.float32) + b2

    x = x_ref[...]
    fu = mlp(x[:, :D])
    fi = mlp(x[:, D:])
    dots = jnp.sum(fu * fi, axis=1)
    err = dots + c_ref[...]
    i = pl.program_id(0)
    part_ref[i, 0] = jnp.sum(err * err)
    part_ref[i, 1] = jnp.sum(fu * fu)
    part_ref[i, 2] = jnp.sum(fi * fi)


def _tc_loss(rows, c, W1, b1, W2, b2):
    return pl.pallas_call(
        _tc_body,
        grid=(NB,),
        in_specs=[
            pl.BlockSpec((TB, 128), lambda i: (i, 0)),
            pl.BlockSpec((TB,), lambda i: (i,)),
            pl.BlockSpec((D, H), lambda i: (0, 0)),
            pl.BlockSpec((1, H), lambda i: (0, 0)),
            pl.BlockSpec((H, D), lambda i: (0, 0)),
            pl.BlockSpec((1, D), lambda i: (0, 0)),
        ],
        out_specs=pl.BlockSpec(memory_space=pltpu.SMEM),
        out_shape=jax.ShapeDtypeStruct((NB, 3), jnp.float32),
    )(rows, c, W1, b1, W2, b2)


def kernel(user0, item_i0, ratings, embed_user, embed_item,
           W1, b1, W2, b2, user_bias, item_bias):
    u0 = user0.astype(jnp.int32)
    i0 = item_i0.astype(jnp.int32)
    t8 = Layout(major_to_minor=(0, 1), tiling=((8,),))
    eu8 = with_layout_constraint(embed_user, t8)
    ei8 = with_layout_constraint(embed_item, t8)
    rows, c = _sc_gather(u0, i0, ratings.astype(jnp.float32), eu8, ei8,
                         user_bias[:, 0], item_bias[:, 0])
    parts = _tc_loss(rows, c, W1, b1.reshape(1, H), W2, b2.reshape(1, D))
    sums = jnp.sum(parts, axis=0)
    loss2 = sums[0] / B
    l2 = LAM * (sums[1] / (B * D)) + LAM * (sums[2] / (B * D))
    loss = loss2 + l2
    z = jnp.float32(0.0)
    return (loss, loss2, z, z, z, z)
